# Initial kernel scaffold; baseline (speedup 1.0000x reference)
import jax
import jax.numpy as jnp
from jax.experimental import pallas as pl


def kernel(x, batch, weight):
    raise NotImplementedError("write your pallas kernel here")


# speedup vs baseline: 4.4184x; 4.4184x over previous
import functools

import jax
import jax.numpy as jnp
from jax import lax
from jax.experimental import pallas as pl
from jax.experimental.pallas import tpu as pltpu
from jax.experimental.pallas import tpu_sc as plsc

_EPS = 1e-6
_NSEG = 256
_BINS = 512
_PUB = 2 * _BINS


def _rowsq_body(x_ref, o_ref):
    xb = x_ref[...]
    o_ref[...] = jnp.sum(xb * xb, axis=1, keepdims=True)


def _apply_body(x_ref, s_ref, w_ref, o_ref):
    o_ref[...] = x_ref[...] * (w_ref[...] * s_ref[...])


@functools.cache
def _make_sc_kernel(n_pad: int, n_per_w: int, nw: int, feat: int):
    nvr = n_per_w // 16
    mesh = plsc.VectorSubcoreMesh(
        core_axis_name="c", subcore_axis_name="s", num_cores=1
    )

    @functools.partial(
        pl.kernel,
        out_type=jax.ShapeDtypeStruct((n_pad,), jnp.float32),
        mesh=mesh,
        compiler_params=pltpu.CompilerParams(needs_layout_passes=False),
        scratch_types=[
            pltpu.VMEM((n_per_w,), jnp.int32),
            pltpu.VMEM((n_per_w,), jnp.float32),
            pltpu.VMEM((n_per_w,), jnp.float32),
            pltpu.VMEM((_PUB,), jnp.float32),
            pltpu.VMEM((_PUB,), jnp.float32),
            pltpu.VMEM((_BINS,), jnp.float32),
            pltpu.VMEM_SHARED((nw, _PUB), jnp.float32),
        ],
    )
    def sc_k(vals_hbm, ids_hbm, out_hbm, ids_v, vals_v, outs_v,
             pub_v, tmp_v, scale_v, sh_all):
        wid = lax.axis_index("s")
        base = wid * n_per_w
        pltpu.sync_copy(ids_hbm.at[pl.ds(base, n_per_w)], ids_v)
        pltpu.sync_copy(vals_hbm.at[pl.ds(base, n_per_w)], vals_v)

        zero16 = jnp.zeros((16,), jnp.float32)
        ones16 = jnp.ones((16,), jnp.float32)

        def zero_body(j, carry):
            pub_v[pl.ds(j * 16, 16)] = zero16
            return carry

        lax.fori_loop(0, _PUB // 16, zero_body, 0)

        def acc_body(j, carry):
            off = j * 16
            idv = ids_v[pl.ds(off, 16)]
            vv = vals_v[pl.ds(off, 16)]
            plsc.addupdate_scatter(pub_v, [idv], vv)
            plsc.addupdate_scatter(pub_v, [idv + _BINS], ones16)
            return carry

        lax.fori_loop(0, nvr, acc_body, 0)

        pltpu.sync_copy(pub_v, sh_all.at[wid])
        plsc.subcore_barrier()

        lax.fori_loop(0, _PUB // 16, zero_body, 0)

        def red_body(t, carry):
            pltpu.sync_copy(sh_all.at[t], tmp_v)

            def add_body(j, c2):
                off = j * 16
                pub_v[pl.ds(off, 16)] = (
                    pub_v[pl.ds(off, 16)] + tmp_v[pl.ds(off, 16)]
                )
                return c2

            lax.fori_loop(0, _PUB // 16, add_body, 0)
            return carry

        lax.fori_loop(0, nw, red_body, 0)

        def scale_body(j, carry):
            off = j * 16
            t = pub_v[pl.ds(off, 16)]
            c = jnp.maximum(pub_v[pl.ds(_BINS + off, 16)], 1.0)
            m = t / (c * float(feat)) + _EPS
            i = lax.bitcast_convert_type(m, jnp.int32)
            i = 0x5F3759DF - lax.shift_right_arithmetic(i, 1)
            y = lax.bitcast_convert_type(i, jnp.float32)
            for _ in range(3):
                y = y * (1.5 - 0.5 * m * y * y)
            scale_v[pl.ds(off, 16)] = y
            return carry

        lax.fori_loop(0, _BINS // 16, scale_body, 0)

        def g_body(j, carry):
            off = j * 16
            idv = ids_v[pl.ds(off, 16)]
            outs_v[pl.ds(off, 16)] = plsc.load_gather(scale_v, [idv])
            return carry

        lax.fori_loop(0, nvr, g_body, 0)
        pltpu.sync_copy(outs_v, out_hbm.at[pl.ds(base, n_per_w)])

    return sc_k


def kernel(x, batch, weight):
    n, feat = x.shape
    ids = batch.astype(jnp.int32)

    nblk = 20
    rb = n // nblk

    rowsq = pl.pallas_call(
        _rowsq_body,
        grid=(nblk,),
        in_specs=[pl.BlockSpec((rb, feat), lambda i: (i, 0))],
        out_specs=pl.BlockSpec((rb, 1), lambda i: (i, 0)),
        out_shape=jax.ShapeDtypeStruct((n, 1), jnp.float32),
    )(x)

    nw = 16
    n_per_w = ((n + nw * 8 - 1) // (nw * 8)) * 8
    n_pad = nw * n_per_w
    vals_p = jnp.pad(rowsq.reshape(n), (0, n_pad - n))
    ids_p = jnp.pad(ids, (0, n_pad - n), constant_values=_NSEG)

    rowscale = _make_sc_kernel(n_pad, n_per_w, nw, feat)(vals_p, ids_p)
    rs = rowscale[:n].reshape(n, 1)

    w2 = weight.reshape(1, feat).astype(jnp.float32)
    out = pl.pallas_call(
        _apply_body,
        grid=(nblk,),
        in_specs=[
            pl.BlockSpec((rb, feat), lambda i: (i, 0)),
            pl.BlockSpec((rb, 1), lambda i: (i, 0)),
            pl.BlockSpec((1, feat), lambda i: (0, 0)),
        ],
        out_specs=pl.BlockSpec((rb, feat), lambda i: (i, 0)),
        out_shape=jax.ShapeDtypeStruct((n, feat), x.dtype),
    )(x, rs, w2)
    return out


# speedup vs baseline: 7.8692x; 1.7810x over previous
import functools

import jax
import jax.numpy as jnp
from jax import lax
from jax.experimental import pallas as pl
from jax.experimental.pallas import tpu as pltpu
from jax.experimental.pallas import tpu_sc as plsc

_EPS = 1e-6
_NSEG = 256


def _partial_body(x_ref, ids_ref, o_ref):
    xb = x_ref[...]
    rowsq = jnp.sum(xb * xb, axis=1, keepdims=True)
    ids = ids_ref[0, 0, :]
    iota = lax.broadcasted_iota(jnp.int32, (1, _NSEG), 1)
    onehot = (ids[:, None] == iota).astype(jnp.float32)
    vals2 = jnp.concatenate(
        [rowsq, jnp.ones_like(rowsq)], axis=1
    )
    part = lax.dot_general(
        vals2, onehot, (((0,), (0,)), ((), ())),
        preferred_element_type=jnp.float32,
    )
    o_ref[0] = part


def _apply_body(x_ref, ids_ref, s_ref, w_ref, o_ref):
    ids = ids_ref[0, 0, :]
    iota = lax.broadcasted_iota(jnp.int32, (1, _NSEG), 1)
    onehot = (ids[:, None] == iota).astype(jnp.float32)
    rowscale = lax.dot_general(
        onehot, s_ref[...], (((1,), (1,)), ((), ())),
        preferred_element_type=jnp.float32,
    )
    o_ref[...] = x_ref[...] * (w_ref[...] * rowscale)


@functools.cache
def _make_sc_reduce(nblk: int, feat: int):
    nchunk = _NSEG // 16
    mesh = plsc.VectorSubcoreMesh(
        core_axis_name="c", subcore_axis_name="s", num_cores=1
    )

    @functools.partial(
        pl.kernel,
        out_type=jax.ShapeDtypeStruct((_NSEG,), jnp.float32),
        mesh=mesh,
        compiler_params=pltpu.CompilerParams(needs_layout_passes=False),
        scratch_types=[
            pltpu.VMEM((nblk, 2, _NSEG), jnp.float32),
            pltpu.VMEM((16,), jnp.float32),
        ],
    )
    def sc_k(part_hbm, out_hbm, parts_v, stage_v):
        tid = lax.axis_index("s")
        off = tid * 16
        pltpu.sync_copy(part_hbm, parts_v)
        sums = jnp.zeros((16,), jnp.float32)
        cnts = jnp.zeros((16,), jnp.float32)
        for b in range(nblk):
            sums = sums + parts_v[b, 0, pl.ds(off, 16)]
            cnts = cnts + parts_v[b, 1, pl.ds(off, 16)]
        m = sums / (jnp.maximum(cnts, 1.0) * float(feat)) + _EPS
        i = lax.bitcast_convert_type(m, jnp.int32)
        i = 0x5F3759DF - lax.shift_right_arithmetic(i, 1)
        y = lax.bitcast_convert_type(i, jnp.float32)
        for _ in range(3):
            y = y * (1.5 - 0.5 * m * y * y)
        stage_v[...] = y
        pltpu.sync_copy(stage_v, out_hbm.at[pl.ds(off, 16)])

    return sc_k


def kernel(x, batch, weight):
    n, feat = x.shape
    nblk = 20
    rb = n // nblk
    ids3 = batch.astype(jnp.int32).reshape(nblk, 1, rb)

    partials = pl.pallas_call(
        _partial_body,
        grid=(nblk,),
        in_specs=[
            pl.BlockSpec((rb, feat), lambda i: (i, 0)),
            pl.BlockSpec((1, 1, rb), lambda i: (i, 0, 0)),
        ],
        out_specs=pl.BlockSpec((1, 2, _NSEG), lambda i: (i, 0, 0)),
        out_shape=jax.ShapeDtypeStruct((nblk, 2, _NSEG), jnp.float32),
    )(x, ids3)

    scale = _make_sc_reduce(nblk, feat)(partials)

    w2 = weight.reshape(1, feat).astype(jnp.float32)
    s2 = scale.reshape(1, _NSEG)
    out = pl.pallas_call(
        _apply_body,
        grid=(nblk,),
        in_specs=[
            pl.BlockSpec((rb, feat), lambda i: (i, 0)),
            pl.BlockSpec((1, 1, rb), lambda i: (i, 0, 0)),
            pl.BlockSpec((1, _NSEG), lambda i: (0, 0)),
            pl.BlockSpec((1, feat), lambda i: (0, 0)),
        ],
        out_specs=pl.BlockSpec((rb, feat), lambda i: (i, 0)),
        out_shape=jax.ShapeDtypeStruct((n, feat), x.dtype),
    )(x, ids3, s2, w2)
    return out
